# Initial kernel scaffold; baseline (speedup 1.0000x reference)
import jax
import jax.numpy as jnp
from jax.experimental import pallas as pl


def kernel(x, conv1, bn1_scale, bn1_shift, L0_0_conv1, L0_0_bn1_scale, L0_0_bn1_shift, L0_0_conv2, L0_0_bn2_scale, L0_0_bn2_shift, L0_1_conv1, L0_1_bn1_scale, L0_1_bn1_shift, L0_1_conv2, L0_1_bn2_scale, L0_1_bn2_shift, L0_2_conv1, L0_2_bn1_scale, L0_2_bn1_shift, L0_2_conv2, L0_2_bn2_scale, L0_2_bn2_shift, L1_0_conv1, L1_0_bn1_scale, L1_0_bn1_shift, L1_0_conv2, L1_0_bn2_scale, L1_0_bn2_shift, L1_0_ds_conv, L1_0_ds_bn_scale, L1_0_ds_bn_shift, L1_1_conv1, L1_1_bn1_scale, L1_1_bn1_shift, L1_1_conv2, L1_1_bn2_scale, L1_1_bn2_shift, L1_2_conv1, L1_2_bn1_scale, L1_2_bn1_shift, L1_2_conv2, L1_2_bn2_scale, L1_2_bn2_shift, L1_3_conv1, L1_3_bn1_scale, L1_3_bn1_shift, L1_3_conv2, L1_3_bn2_scale, L1_3_bn2_shift, L2_0_conv1, L2_0_bn1_scale, L2_0_bn1_shift, L2_0_conv2, L2_0_bn2_scale, L2_0_bn2_shift, L2_0_ds_conv, L2_0_ds_bn_scale, L2_0_ds_bn_shift, L2_1_conv1, L2_1_bn1_scale, L2_1_bn1_shift, L2_1_conv2, L2_1_bn2_scale, L2_1_bn2_shift, L2_2_conv1, L2_2_bn1_scale, L2_2_bn1_shift, L2_2_conv2, L2_2_bn2_scale, L2_2_bn2_shift, L2_3_conv1, L2_3_bn1_scale, L2_3_bn1_shift, L2_3_conv2, L2_3_bn2_scale, L2_3_bn2_shift, L2_4_conv1, L2_4_bn1_scale, L2_4_bn1_shift, L2_4_conv2, L2_4_bn2_scale, L2_4_bn2_shift, L2_5_conv1, L2_5_bn1_scale, L2_5_bn1_shift, L2_5_conv2, L2_5_bn2_scale, L2_5_bn2_shift, L3_0_conv1, L3_0_bn1_scale, L3_0_bn1_shift, L3_0_conv2, L3_0_bn2_scale, L3_0_bn2_shift, L3_0_ds_conv, L3_0_ds_bn_scale, L3_0_ds_bn_shift, L3_1_conv1, L3_1_bn1_scale, L3_1_bn1_shift, L3_1_conv2, L3_1_bn2_scale, L3_1_bn2_shift, L3_2_conv1, L3_2_bn1_scale, L3_2_bn1_shift, L3_2_conv2, L3_2_bn2_scale, L3_2_bn2_shift, scores1_w, scores1_b, scores2_w, scores2_b, scores3_w, scores3_b, upsample_8x, upsample_4x, upsample_2x):
    raise NotImplementedError("write your pallas kernel here")


# speedup vs baseline: 2.1310x; 2.1310x over previous
import functools

import jax
import jax.numpy as jnp
from jax.experimental import pallas as pl
from jax.experimental.pallas import tpu as pltpu

BF = jnp.bfloat16
F32 = jnp.float32


def _w3(w):
    co, ci, _, _ = w.shape
    return jnp.transpose(w, (2, 3, 1, 0)).reshape(9, ci, co).astype(BF)


def _w1(w):
    co, ci, _, _ = w.shape
    return jnp.transpose(w.reshape(co, ci), (1, 0)).astype(BF)


def _row(v):
    return v.reshape(1, -1).astype(F32)


def _phases(x, pad=1):
    out = []
    for pr in range(2):
        for pc in range(2):
            p = x[:, pr::2, pc::2, :]
            out.append(jnp.pad(p, ((0, 0), (pad, pad), (pad, pad), (0, 0))))
    return out


def _conv3x3(xp, w, OH, OW):
    Cin = xp.shape[2]
    acc = None
    for i in range(3):
        for j in range(3):
            t = jax.lax.slice(xp, (i, j, 0), (i + OH, j + OW, Cin))
            d = jnp.dot(t.reshape(OH * OW, Cin), w[3 * i + j],
                        preferred_element_type=F32)
            acc = d if acc is None else acc + d
    return acc


def _conv3x3_s2(pp, w, OH, OW):
    Cin = pp[0].shape[2]
    acc = None
    for i in range(3):
        pr, a = (i - 1) % 2, (i + 1) // 2
        for j in range(3):
            pc, b = (j - 1) % 2, (j + 1) // 2
            src = pp[2 * pr + pc]
            t = jax.lax.slice(src, (a, b, 0), (a + OH, b + OW, Cin))
            d = jnp.dot(t.reshape(OH * OW, Cin), w[3 * i + j],
                        preferred_element_type=F32)
            acc = d if acc is None else acc + d
    return acc


def _res_block_tail(x_f32_flat, y, w2, s2, h2, H, W, C):
    yp = jnp.pad(y.astype(BF).reshape(H, W, C), ((1, 1), (1, 1), (0, 0)))
    z = _conv3x3(yp, w2, H, W)
    z = z * s2 + h2 + x_f32_flat
    return jnp.maximum(z, 0.0).reshape(H, W, C)


def _rb_kernel(x_ref, w1_ref, s1_ref, h1_ref, w2_ref, s2_ref, h2_ref, o_ref,
               *, H, W, C):
    x = x_ref[0]
    xp = jnp.pad(x.astype(BF), ((1, 1), (1, 1), (0, 0)))
    y = _conv3x3(xp, w1_ref[...], H, W)
    y = jnp.maximum(y * s1_ref[...] + h1_ref[...], 0.0)
    o_ref[0] = _res_block_tail(x.reshape(H * W, C), y, w2_ref[...],
                               s2_ref[...], h2_ref[...], H, W, C)


def _rbp_kernel(p0_ref, p1_ref, p2_ref, p3_ref,
                w1_ref, s1_ref, h1_ref, w2_ref, s2_ref, h2_ref, o_ref,
                *, H, W, C):
    pp = [p0_ref[0], p1_ref[0], p2_ref[0], p3_ref[0]]
    m = None
    for i in range(3):
        pr, a = (i - 1) % 2, (i + 1) // 2
        for j in range(3):
            pc, b = (j - 1) % 2, (j + 1) // 2
            t = jax.lax.slice(pp[2 * pr + pc], (a, b, 0), (a + H, b + W, C))
            m = t if m is None else jnp.maximum(m, t)
    x = m
    xp = jnp.pad(x.astype(BF), ((1, 1), (1, 1), (0, 0)))
    y = _conv3x3(xp, w1_ref[...], H, W)
    y = jnp.maximum(y * s1_ref[...] + h1_ref[...], 0.0)
    o_ref[0] = _res_block_tail(x.reshape(H * W, C), y, w2_ref[...],
                               s2_ref[...], h2_ref[...], H, W, C)


def _rbd_kernel(p0_ref, p1_ref, p2_ref, p3_ref,
                w1_ref, s1_ref, h1_ref, w2_ref, s2_ref, h2_ref,
                wd_ref, sd_ref, hd_ref, o_ref, *, H, W, C):
    pp = [p0_ref[0], p1_ref[0], p2_ref[0], p3_ref[0]]
    Cin = pp[0].shape[2]
    y = _conv3x3_s2(pp, w1_ref[...], H, W)
    y = jnp.maximum(y * s1_ref[...] + h1_ref[...], 0.0)
    yp = jnp.pad(y.astype(BF).reshape(H, W, C), ((1, 1), (1, 1), (0, 0)))
    z = _conv3x3(yp, w2_ref[...], H, W)
    xs = jax.lax.slice(pp[0], (1, 1, 0), (1 + H, 1 + W, Cin))
    idn = jnp.dot(xs.reshape(H * W, Cin), wd_ref[...],
                  preferred_element_type=F32)
    idn = idn * sd_ref[...] + hd_ref[...]
    z = z * s2_ref[...] + h2_ref[...] + idn
    o_ref[0] = jnp.maximum(z, 0.0).reshape(H, W, C)


def _pcall(body, ins, specs, N, H, W, C):
    return pl.pallas_call(
        body,
        out_shape=jax.ShapeDtypeStruct((N, H, W, C), F32),
        grid_spec=pltpu.PrefetchScalarGridSpec(
            num_scalar_prefetch=0,
            grid=(N,),
            in_specs=specs,
            out_specs=pl.BlockSpec((1, H, W, C), lambda n: (n, 0, 0, 0)),
        ),
        compiler_params=pltpu.CompilerParams(
            dimension_semantics=("parallel",)),
    )(*ins)


def _wspecs(Cin, C):
    return [
        pl.BlockSpec((9, Cin, C), lambda n: (0, 0, 0)),
        pl.BlockSpec((1, C), lambda n: (0, 0)),
        pl.BlockSpec((1, C), lambda n: (0, 0)),
        pl.BlockSpec((9, C, C), lambda n: (0, 0, 0)),
        pl.BlockSpec((1, C), lambda n: (0, 0)),
        pl.BlockSpec((1, C), lambda n: (0, 0)),
    ]


def _block(x, w1, s1, h1, w2, s2, h2):
    N, H, W, C = x.shape
    ins = [x, _w3(w1), _row(s1), _row(h1), _w3(w2), _row(s2), _row(h2)]
    specs = ([pl.BlockSpec((1, H, W, C), lambda n: (n, 0, 0, 0))]
             + _wspecs(C, C))
    body = functools.partial(_rb_kernel, H=H, W=W, C=C)
    return _pcall(body, ins, specs, N, H, W, C)


def _block_pool(x, w1, s1, h1, w2, s2, h2):
    N, Hin, Win, C = x.shape
    H, W = Hin // 2, Win // 2
    ph = _phases(x)
    ins = ph + [_w3(w1), _row(s1), _row(h1), _w3(w2), _row(s2), _row(h2)]
    pspec = pl.BlockSpec((1, H + 2, W + 2, C), lambda n: (n, 0, 0, 0))
    specs = [pspec] * 4 + _wspecs(C, C)
    body = functools.partial(_rbp_kernel, H=H, W=W, C=C)
    return _pcall(body, ins, specs, N, H, W, C)


def _block_ds(x, w1, s1, h1, w2, s2, h2, wd, sd, hd):
    N, Hin, Win, Cin = x.shape
    C = w1.shape[0]
    H, W = Hin // 2, Win // 2
    ph = _phases(x.astype(BF))
    ins = (ph + [_w3(w1), _row(s1), _row(h1), _w3(w2), _row(s2), _row(h2),
                 _w1(wd), _row(sd), _row(hd)])
    pspec = pl.BlockSpec((1, H + 2, W + 2, Cin), lambda n: (n, 0, 0, 0))
    specs = ([pspec] * 4 + _wspecs(Cin, C)
             + [pl.BlockSpec((Cin, C), lambda n: (0, 0)),
                pl.BlockSpec((1, C), lambda n: (0, 0)),
                pl.BlockSpec((1, C), lambda n: (0, 0))])
    body = functools.partial(_rbd_kernel, H=H, W=W, C=C)
    return _pcall(body, ins, specs, N, H, W, C)


def _stem_kernel(p_ref, w_ref, s_ref, h_ref, o_ref):
    y = jnp.dot(p_ref[0], w_ref[...], preferred_element_type=F32)
    y = jnp.maximum(y * s_ref[...] + h_ref[...], 0.0)
    o_ref[0] = y.reshape(128, 128, 64)


def _stem(x, conv1, s, h):
    N = x.shape[0]
    xb = jnp.transpose(x, (0, 2, 3, 1)).astype(BF)
    xp = jnp.pad(xb, ((0, 0), (3, 3), (3, 3), (0, 0)))
    cols = [xp[:, i:i + 255:2, j:j + 255:2, :]
            for i in range(7) for j in range(7)]
    pat = jnp.stack(cols, axis=3).reshape(N, 128 * 128, 147)
    w = jnp.transpose(conv1, (2, 3, 1, 0)).reshape(147, 64).astype(BF)
    return pl.pallas_call(
        _stem_kernel,
        out_shape=jax.ShapeDtypeStruct((N, 128, 128, 64), F32),
        grid_spec=pltpu.PrefetchScalarGridSpec(
            num_scalar_prefetch=0,
            grid=(N,),
            in_specs=[
                pl.BlockSpec((1, 128 * 128, 147), lambda n: (n, 0, 0)),
                pl.BlockSpec((147, 64), lambda n: (0, 0)),
                pl.BlockSpec((1, 64), lambda n: (0, 0)),
                pl.BlockSpec((1, 64), lambda n: (0, 0)),
            ],
            out_specs=pl.BlockSpec((1, 128, 128, 64), lambda n: (n, 0, 0, 0)),
        ),
        compiler_params=pltpu.CompilerParams(
            dimension_semantics=("parallel",)),
    )(pat, w, _row(s), _row(h))


def _mm_kernel(a_ref, b_ref, h_ref, *rest, has_res):
    if has_res:
        res_ref, o_ref = rest
    else:
        (o_ref,) = rest
    y = jnp.dot(a_ref[0], b_ref[...], preferred_element_type=F32)
    y = y + h_ref[...]
    if has_res:
        y = y + res_ref[0]
    o_ref[0] = y


def _mm(a, b, shift=None, res=None):
    N, M, K = a.shape
    Nc = b.shape[1]
    if shift is None:
        shift = jnp.zeros((1, Nc), F32)
    ins = [a, b, shift]
    specs = [
        pl.BlockSpec((1, M, K), lambda n: (n, 0, 0)),
        pl.BlockSpec((K, Nc), lambda n: (0, 0)),
        pl.BlockSpec((1, Nc), lambda n: (0, 0)),
    ]
    if res is not None:
        ins.append(res)
        specs.append(pl.BlockSpec((1, M, Nc), lambda n: (n, 0, 0)))
    return pl.pallas_call(
        functools.partial(_mm_kernel, has_res=res is not None),
        out_shape=jax.ShapeDtypeStruct((N, M, Nc), F32),
        grid_spec=pltpu.PrefetchScalarGridSpec(
            num_scalar_prefetch=0,
            grid=(N,),
            in_specs=specs,
            out_specs=pl.BlockSpec((1, M, Nc), lambda n: (n, 0, 0)),
        ),
        compiler_params=pltpu.CompilerParams(
            dimension_semantics=("parallel",)),
    )(*ins)


def _score(x, w, b, res=None):
    N, H, W, Cin = x.shape
    Nc = w.shape[0]
    a = x.astype(BF).reshape(N, H * W, Cin)
    r = None if res is None else res.reshape(N, H * W, Nc)
    out = _mm(a, _w1(w), _row(b), r)
    return out.reshape(N, H, W, Nc)


def _upsample(x, w, s, pad):
    Cin, Cout = w.shape[0], w.shape[1]
    N, H, W, _ = x.shape
    xp = jnp.pad(x.astype(BF), ((0, 0), (1, 1), (1, 1), (0, 0)))
    cols = [xp[:, a:a + H + 1, b:b + W + 1, :]
            for a in range(2) for b in range(2)]
    pat = jnp.stack(cols, axis=3).reshape(N, (H + 1) * (W + 1), 4 * Cin)
    wk = w.reshape(Cin, Cout, 2, s, 2, s)[:, :, ::-1, :, ::-1, :]
    wmat = jnp.transpose(wk, (2, 4, 0, 3, 5, 1)).reshape(
        4 * Cin, s * s * Cout).astype(BF)
    out = _mm(pat, wmat)
    full = out.reshape(N, H + 1, W + 1, s, s, Cout)
    full = jnp.transpose(full, (0, 1, 3, 2, 4, 5)).reshape(
        N, (H + 1) * s, (W + 1) * s, Cout)
    oh = (H + 1) * s - 2 * pad
    ow = (W + 1) * s - 2 * pad
    return full[:, pad:pad + oh, pad:pad + ow, :]


def kernel(x, conv1, bn1_scale, bn1_shift, L0_0_conv1, L0_0_bn1_scale, L0_0_bn1_shift, L0_0_conv2, L0_0_bn2_scale, L0_0_bn2_shift, L0_1_conv1, L0_1_bn1_scale, L0_1_bn1_shift, L0_1_conv2, L0_1_bn2_scale, L0_1_bn2_shift, L0_2_conv1, L0_2_bn1_scale, L0_2_bn1_shift, L0_2_conv2, L0_2_bn2_scale, L0_2_bn2_shift, L1_0_conv1, L1_0_bn1_scale, L1_0_bn1_shift, L1_0_conv2, L1_0_bn2_scale, L1_0_bn2_shift, L1_0_ds_conv, L1_0_ds_bn_scale, L1_0_ds_bn_shift, L1_1_conv1, L1_1_bn1_scale, L1_1_bn1_shift, L1_1_conv2, L1_1_bn2_scale, L1_1_bn2_shift, L1_2_conv1, L1_2_bn1_scale, L1_2_bn1_shift, L1_2_conv2, L1_2_bn2_scale, L1_2_bn2_shift, L1_3_conv1, L1_3_bn1_scale, L1_3_bn1_shift, L1_3_conv2, L1_3_bn2_scale, L1_3_bn2_shift, L2_0_conv1, L2_0_bn1_scale, L2_0_bn1_shift, L2_0_conv2, L2_0_bn2_scale, L2_0_bn2_shift, L2_0_ds_conv, L2_0_ds_bn_scale, L2_0_ds_bn_shift, L2_1_conv1, L2_1_bn1_scale, L2_1_bn1_shift, L2_1_conv2, L2_1_bn2_scale, L2_1_bn2_shift, L2_2_conv1, L2_2_bn1_scale, L2_2_bn1_shift, L2_2_conv2, L2_2_bn2_scale, L2_2_bn2_shift, L2_3_conv1, L2_3_bn1_scale, L2_3_bn1_shift, L2_3_conv2, L2_3_bn2_scale, L2_3_bn2_shift, L2_4_conv1, L2_4_bn1_scale, L2_4_bn1_shift, L2_4_conv2, L2_4_bn2_scale, L2_4_bn2_shift, L2_5_conv1, L2_5_bn1_scale, L2_5_bn1_shift, L2_5_conv2, L2_5_bn2_scale, L2_5_bn2_shift, L3_0_conv1, L3_0_bn1_scale, L3_0_bn1_shift, L3_0_conv2, L3_0_bn2_scale, L3_0_bn2_shift, L3_0_ds_conv, L3_0_ds_bn_scale, L3_0_ds_bn_shift, L3_1_conv1, L3_1_bn1_scale, L3_1_bn1_shift, L3_1_conv2, L3_1_bn2_scale, L3_1_bn2_shift, L3_2_conv1, L3_2_bn1_scale, L3_2_bn1_shift, L3_2_conv2, L3_2_bn2_scale, L3_2_bn2_shift, scores1_w, scores1_b, scores2_w, scores2_b, scores3_w, scores3_b, upsample_8x, upsample_4x, upsample_2x):
    h = _stem(x, conv1, bn1_scale, bn1_shift)

    h = _block_pool(h, L0_0_conv1, L0_0_bn1_scale, L0_0_bn1_shift,
                    L0_0_conv2, L0_0_bn2_scale, L0_0_bn2_shift)
    h = _block(h, L0_1_conv1, L0_1_bn1_scale, L0_1_bn1_shift,
               L0_1_conv2, L0_1_bn2_scale, L0_1_bn2_shift)
    h = _block(h, L0_2_conv1, L0_2_bn1_scale, L0_2_bn1_shift,
               L0_2_conv2, L0_2_bn2_scale, L0_2_bn2_shift)

    h = _block_ds(h, L1_0_conv1, L1_0_bn1_scale, L1_0_bn1_shift, L1_0_conv2,
                  L1_0_bn2_scale, L1_0_bn2_shift,
                  L1_0_ds_conv, L1_0_ds_bn_scale, L1_0_ds_bn_shift)
    for blk in [
        (L1_1_conv1, L1_1_bn1_scale, L1_1_bn1_shift, L1_1_conv2, L1_1_bn2_scale, L1_1_bn2_shift),
        (L1_2_conv1, L1_2_bn1_scale, L1_2_bn1_shift, L1_2_conv2, L1_2_bn2_scale, L1_2_bn2_shift),
        (L1_3_conv1, L1_3_bn1_scale, L1_3_bn1_shift, L1_3_conv2, L1_3_bn2_scale, L1_3_bn2_shift),
    ]:
        h = _block(h, *blk)
    s1 = h

    h = _block_ds(h, L2_0_conv1, L2_0_bn1_scale, L2_0_bn1_shift, L2_0_conv2,
                  L2_0_bn2_scale, L2_0_bn2_shift,
                  L2_0_ds_conv, L2_0_ds_bn_scale, L2_0_ds_bn_shift)
    for blk in [
        (L2_1_conv1, L2_1_bn1_scale, L2_1_bn1_shift, L2_1_conv2, L2_1_bn2_scale, L2_1_bn2_shift),
        (L2_2_conv1, L2_2_bn1_scale, L2_2_bn1_shift, L2_2_conv2, L2_2_bn2_scale, L2_2_bn2_shift),
        (L2_3_conv1, L2_3_bn1_scale, L2_3_bn1_shift, L2_3_conv2, L2_3_bn2_scale, L2_3_bn2_shift),
        (L2_4_conv1, L2_4_bn1_scale, L2_4_bn1_shift, L2_4_conv2, L2_4_bn2_scale, L2_4_bn2_shift),
        (L2_5_conv1, L2_5_bn1_scale, L2_5_bn1_shift, L2_5_conv2, L2_5_bn2_scale, L2_5_bn2_shift),
    ]:
        h = _block(h, *blk)
    s2 = h

    h = _block_ds(h, L3_0_conv1, L3_0_bn1_scale, L3_0_bn1_shift, L3_0_conv2,
                  L3_0_bn2_scale, L3_0_bn2_shift,
                  L3_0_ds_conv, L3_0_ds_bn_scale, L3_0_ds_bn_shift)
    for blk in [
        (L3_1_conv1, L3_1_bn1_scale, L3_1_bn1_shift, L3_1_conv2, L3_1_bn2_scale, L3_1_bn2_shift),
        (L3_2_conv1, L3_2_bn1_scale, L3_2_bn1_shift, L3_2_conv2, L3_2_bn2_scale, L3_2_bn2_shift),
    ]:
        h = _block(h, *blk)
    s3 = h

    t3 = _score(s3, scores1_w, scores1_b)
    t3 = _upsample(t3, upsample_2x, 2, 1)
    t2 = _score(s2, scores2_w, scores2_b, res=t3)
    t2 = _upsample(t2, upsample_4x, 2, 1)
    t1 = _score(s1, scores3_w, scores3_b, res=t2)
    out = _upsample(t1, upsample_8x, 8, 4)
    return jnp.transpose(out, (0, 3, 1, 2))


# speedup vs baseline: 6.6822x; 3.1358x over previous
import functools

import jax
import jax.numpy as jnp
from jax.experimental import pallas as pl
from jax.experimental.pallas import tpu as pltpu

BF = jnp.bfloat16
F32 = jnp.float32


def _w3(w):
    co, ci, _, _ = w.shape
    return jnp.transpose(w, (2, 3, 1, 0)).reshape(9, ci, co).astype(BF)


def _w1(w):
    co, ci, _, _ = w.shape
    return jnp.transpose(w.reshape(co, ci), (1, 0)).astype(BF)


def _row(v):
    return v.reshape(1, -1).astype(F32)


def _phases_t(x):
    N, H2, W2, C = x.shape
    H, W = H2 // 2, W2 // 2
    t = x.reshape(N, H, 2, W, 2, C).transpose(0, 2, 4, 1, 3, 5)
    return t.reshape(N, 4, H, W, C)


def _conv3x3(xp, w, OH, OW):
    Cin = xp.shape[2]
    acc = None
    for i in range(3):
        for j in range(3):
            t = jax.lax.slice(xp, (i, j, 0), (i + OH, j + OW, Cin))
            d = jnp.dot(t.reshape(OH * OW, Cin), w[3 * i + j],
                        preferred_element_type=F32)
            acc = d if acc is None else acc + d
    return acc


def _conv3x3_s2(pp, w, OH, OW):
    Cin = pp[0].shape[2]
    acc = None
    for i in range(3):
        pr, a = (i - 1) % 2, (i + 1) // 2
        for j in range(3):
            pc, b = (j - 1) % 2, (j + 1) // 2
            src = pp[2 * pr + pc]
            t = jax.lax.slice(src, (a, b, 0), (a + OH, b + OW, Cin))
            d = jnp.dot(t.reshape(OH * OW, Cin), w[3 * i + j],
                        preferred_element_type=F32)
            acc = d if acc is None else acc + d
    return acc


def _res_block_tail(x_f32_flat, y, w2, s2, h2, H, W, C):
    yp = jnp.pad(y.astype(BF).reshape(H, W, C), ((1, 1), (1, 1), (0, 0)))
    z = _conv3x3(yp, w2, H, W)
    z = z * s2 + h2 + x_f32_flat
    return jnp.maximum(z, 0.0).reshape(H, W, C)


def _rb_kernel(x_ref, w1_ref, s1_ref, h1_ref, w2_ref, s2_ref, h2_ref, o_ref,
               *, H, W, C):
    x = x_ref[0]
    xp = jnp.pad(x.astype(BF), ((1, 1), (1, 1), (0, 0)))
    y = _conv3x3(xp, w1_ref[...], H, W)
    y = jnp.maximum(y * s1_ref[...] + h1_ref[...], 0.0)
    o_ref[0] = _res_block_tail(x.reshape(H * W, C), y, w2_ref[...],
                               s2_ref[...], h2_ref[...], H, W, C)


def _stem_rb_kernel(s4_ref, w00_ref, w01_ref, w10_ref, w11_ref,
                    ss_ref, sh_ref,
                    w1_ref, s1_ref, h1_ref, w2_ref, s2_ref, h2_ref, o_ref,
                    *, H, W, C):
    s4 = s4_ref[0]
    ys = []
    for wref in (w00_ref, w01_ref, w10_ref, w11_ref):
        y = _conv3x3(s4, wref[...], H, W)
        y = jnp.maximum(y * ss_ref[...] + sh_ref[...], 0.0)
        ys.append(jnp.pad(y.reshape(H, W, C), ((1, 1), (1, 1), (0, 0))))
    m = None
    for i in range(3):
        pr, a = (i - 1) % 2, (i + 1) // 2
        for j in range(3):
            pc, b = (j - 1) % 2, (j + 1) // 2
            t = jax.lax.slice(ys[2 * pr + pc], (a, b, 0), (a + H, b + W, C))
            m = t if m is None else jnp.maximum(m, t)
    x = m
    xp = jnp.pad(x.astype(BF), ((1, 1), (1, 1), (0, 0)))
    y = _conv3x3(xp, w1_ref[...], H, W)
    y = jnp.maximum(y * s1_ref[...] + h1_ref[...], 0.0)
    o_ref[0] = _res_block_tail(x.reshape(H * W, C), y, w2_ref[...],
                               s2_ref[...], h2_ref[...], H, W, C)


def _rbd_kernel(p_ref, w1_ref, s1_ref, h1_ref, w2_ref, s2_ref, h2_ref,
                wd_ref, sd_ref, hd_ref, o_ref, *, H, W, C):
    pp = [jnp.pad(p_ref[0, k], ((1, 1), (1, 1), (0, 0))) for k in range(4)]
    Cin = pp[0].shape[2]
    y = _conv3x3_s2(pp, w1_ref[...], H, W)
    y = jnp.maximum(y * s1_ref[...] + h1_ref[...], 0.0)
    yp = jnp.pad(y.astype(BF).reshape(H, W, C), ((1, 1), (1, 1), (0, 0)))
    z = _conv3x3(yp, w2_ref[...], H, W)
    xs = jax.lax.slice(pp[0], (1, 1, 0), (1 + H, 1 + W, Cin))
    idn = jnp.dot(xs.reshape(H * W, Cin), wd_ref[...],
                  preferred_element_type=F32)
    idn = idn * sd_ref[...] + hd_ref[...]
    z = z * s2_ref[...] + h2_ref[...] + idn
    o_ref[0] = jnp.maximum(z, 0.0).reshape(H, W, C)


def _pcall(body, ins, specs, N, H, W, C):
    return pl.pallas_call(
        body,
        out_shape=jax.ShapeDtypeStruct((N, H, W, C), F32),
        grid_spec=pltpu.PrefetchScalarGridSpec(
            num_scalar_prefetch=0,
            grid=(N,),
            in_specs=specs,
            out_specs=pl.BlockSpec((1, H, W, C), lambda n: (n, 0, 0, 0)),
        ),
        compiler_params=pltpu.CompilerParams(
            dimension_semantics=("parallel",)),
    )(*ins)


def _wspecs(Cin, C):
    return [
        pl.BlockSpec((9, Cin, C), lambda n: (0, 0, 0)),
        pl.BlockSpec((1, C), lambda n: (0, 0)),
        pl.BlockSpec((1, C), lambda n: (0, 0)),
        pl.BlockSpec((9, C, C), lambda n: (0, 0, 0)),
        pl.BlockSpec((1, C), lambda n: (0, 0)),
        pl.BlockSpec((1, C), lambda n: (0, 0)),
    ]


def _block(x, w1, s1, h1, w2, s2, h2):
    N, H, W, C = x.shape
    ins = [x, _w3(w1), _row(s1), _row(h1), _w3(w2), _row(s2), _row(h2)]
    specs = ([pl.BlockSpec((1, H, W, C), lambda n: (n, 0, 0, 0))]
             + _wspecs(C, C))
    body = functools.partial(_rb_kernel, H=H, W=W, C=C)
    return _pcall(body, ins, specs, N, H, W, C)


def _block_ds(x, w1, s1, h1, w2, s2, h2, wd, sd, hd):
    N, Hin, Win, Cin = x.shape
    C = w1.shape[0]
    H, W = Hin // 2, Win // 2
    ph = _phases_t(x.astype(BF))
    ins = [ph, _w3(w1), _row(s1), _row(h1), _w3(w2), _row(s2), _row(h2),
           _w1(wd), _row(sd), _row(hd)]
    pspec = pl.BlockSpec((1, 4, H, W, Cin), lambda n: (n, 0, 0, 0, 0))
    specs = ([pspec] + _wspecs(Cin, C)
             + [pl.BlockSpec((Cin, C), lambda n: (0, 0)),
                pl.BlockSpec((1, C), lambda n: (0, 0)),
                pl.BlockSpec((1, C), lambda n: (0, 0))])
    body = functools.partial(_rbd_kernel, H=H, W=W, C=C)
    return _pcall(body, ins, specs, N, H, W, C)


def _stem_block(x, conv1, s, h, w1, s1, h1, w2, s2, h2):
    N = x.shape[0]
    s4 = x.astype(BF).reshape(N, 3, 64, 4, 64, 4)
    s4 = s4.transpose(0, 2, 4, 3, 5, 1).reshape(N, 64, 64, 48)
    s4 = jnp.pad(s4, ((0, 0), (1, 1), (1, 1), (0, 0)))

    wps = []
    for p in range(2):
        for q in range(2):
            w12 = jnp.pad(conv1, ((0, 0), (0, 0),
                                  (2 * p + 1, 4 - 2 * p),
                                  (2 * q + 1, 4 - 2 * q)))
            w12 = w12.reshape(64, 3, 3, 4, 3, 4)
            w12 = jnp.transpose(w12, (2, 4, 3, 5, 1, 0))
            wps.append(w12.reshape(9, 48, 64).astype(BF))

    ins = [s4] + wps + [_row(s), _row(h),
                        _w3(w1), _row(s1), _row(h1),
                        _w3(w2), _row(s2), _row(h2)]
    wp_spec = pl.BlockSpec((9, 48, 64), lambda n: (0, 0, 0))
    specs = ([pl.BlockSpec((1, 66, 66, 48), lambda n: (n, 0, 0, 0))]
             + [wp_spec] * 4
             + [pl.BlockSpec((1, 64), lambda n: (0, 0))] * 2
             + _wspecs(64, 64))
    body = functools.partial(_stem_rb_kernel, H=64, W=64, C=64)
    return _pcall(body, ins, specs, N, 64, 64, 64)


def _mm_kernel(a_ref, b_ref, h_ref, *rest, has_res):
    if has_res:
        res_ref, o_ref = rest
    else:
        (o_ref,) = rest
    y = jnp.dot(a_ref[0], b_ref[...], preferred_element_type=F32)
    y = y + h_ref[...]
    if has_res:
        y = y + res_ref[0]
    o_ref[0] = y


def _mm(a, b, shift=None, res=None):
    N, M, K = a.shape
    Nc = b.shape[1]
    if shift is None:
        shift = jnp.zeros((1, Nc), F32)
    ins = [a, b, shift]
    specs = [
        pl.BlockSpec((1, M, K), lambda n: (n, 0, 0)),
        pl.BlockSpec((K, Nc), lambda n: (0, 0)),
        pl.BlockSpec((1, Nc), lambda n: (0, 0)),
    ]
    if res is not None:
        ins.append(res)
        specs.append(pl.BlockSpec((1, M, Nc), lambda n: (n, 0, 0)))
    return pl.pallas_call(
        functools.partial(_mm_kernel, has_res=res is not None),
        out_shape=jax.ShapeDtypeStruct((N, M, Nc), F32),
        grid_spec=pltpu.PrefetchScalarGridSpec(
            num_scalar_prefetch=0,
            grid=(N,),
            in_specs=specs,
            out_specs=pl.BlockSpec((1, M, Nc), lambda n: (n, 0, 0)),
        ),
        compiler_params=pltpu.CompilerParams(
            dimension_semantics=("parallel",)),
    )(*ins)


def _score(x, w, b, res=None):
    N, H, W, Cin = x.shape
    Nc = w.shape[0]
    a = x.astype(BF).reshape(N, H * W, Cin)
    r = None if res is None else res.reshape(N, H * W, Nc)
    out = _mm(a, _w1(w), _row(b), r)
    return out.reshape(N, H, W, Nc)


def _upsample(x, w, s, pad):
    Cin, Cout = w.shape[0], w.shape[1]
    N, H, W, _ = x.shape
    xp = jnp.pad(x.astype(BF), ((0, 0), (1, 1), (1, 1), (0, 0)))
    cols = [xp[:, a:a + H + 1, b:b + W + 1, :]
            for a in range(2) for b in range(2)]
    pat = jnp.stack(cols, axis=3).reshape(N, (H + 1) * (W + 1), 4 * Cin)
    wk = w.reshape(Cin, Cout, 2, s, 2, s)[:, :, ::-1, :, ::-1, :]
    wmat = jnp.transpose(wk, (2, 4, 0, 3, 5, 1)).reshape(
        4 * Cin, s * s * Cout).astype(BF)
    out = _mm(pat, wmat)
    full = out.reshape(N, H + 1, W + 1, s, s, Cout)
    full = jnp.transpose(full, (0, 1, 3, 2, 4, 5)).reshape(
        N, (H + 1) * s, (W + 1) * s, Cout)
    oh = (H + 1) * s - 2 * pad
    ow = (W + 1) * s - 2 * pad
    return full[:, pad:pad + oh, pad:pad + ow, :]


def kernel(x, conv1, bn1_scale, bn1_shift, L0_0_conv1, L0_0_bn1_scale, L0_0_bn1_shift, L0_0_conv2, L0_0_bn2_scale, L0_0_bn2_shift, L0_1_conv1, L0_1_bn1_scale, L0_1_bn1_shift, L0_1_conv2, L0_1_bn2_scale, L0_1_bn2_shift, L0_2_conv1, L0_2_bn1_scale, L0_2_bn1_shift, L0_2_conv2, L0_2_bn2_scale, L0_2_bn2_shift, L1_0_conv1, L1_0_bn1_scale, L1_0_bn1_shift, L1_0_conv2, L1_0_bn2_scale, L1_0_bn2_shift, L1_0_ds_conv, L1_0_ds_bn_scale, L1_0_ds_bn_shift, L1_1_conv1, L1_1_bn1_scale, L1_1_bn1_shift, L1_1_conv2, L1_1_bn2_scale, L1_1_bn2_shift, L1_2_conv1, L1_2_bn1_scale, L1_2_bn1_shift, L1_2_conv2, L1_2_bn2_scale, L1_2_bn2_shift, L1_3_conv1, L1_3_bn1_scale, L1_3_bn1_shift, L1_3_conv2, L1_3_bn2_scale, L1_3_bn2_shift, L2_0_conv1, L2_0_bn1_scale, L2_0_bn1_shift, L2_0_conv2, L2_0_bn2_scale, L2_0_bn2_shift, L2_0_ds_conv, L2_0_ds_bn_scale, L2_0_ds_bn_shift, L2_1_conv1, L2_1_bn1_scale, L2_1_bn1_shift, L2_1_conv2, L2_1_bn2_scale, L2_1_bn2_shift, L2_2_conv1, L2_2_bn1_scale, L2_2_bn1_shift, L2_2_conv2, L2_2_bn2_scale, L2_2_bn2_shift, L2_3_conv1, L2_3_bn1_scale, L2_3_bn1_shift, L2_3_conv2, L2_3_bn2_scale, L2_3_bn2_shift, L2_4_conv1, L2_4_bn1_scale, L2_4_bn1_shift, L2_4_conv2, L2_4_bn2_scale, L2_4_bn2_shift, L2_5_conv1, L2_5_bn1_scale, L2_5_bn1_shift, L2_5_conv2, L2_5_bn2_scale, L2_5_bn2_shift, L3_0_conv1, L3_0_bn1_scale, L3_0_bn1_shift, L3_0_conv2, L3_0_bn2_scale, L3_0_bn2_shift, L3_0_ds_conv, L3_0_ds_bn_scale, L3_0_ds_bn_shift, L3_1_conv1, L3_1_bn1_scale, L3_1_bn1_shift, L3_1_conv2, L3_1_bn2_scale, L3_1_bn2_shift, L3_2_conv1, L3_2_bn1_scale, L3_2_bn1_shift, L3_2_conv2, L3_2_bn2_scale, L3_2_bn2_shift, scores1_w, scores1_b, scores2_w, scores2_b, scores3_w, scores3_b, upsample_8x, upsample_4x, upsample_2x):
    h = _stem_block(x, conv1, bn1_scale, bn1_shift,
                    L0_0_conv1, L0_0_bn1_scale, L0_0_bn1_shift,
                    L0_0_conv2, L0_0_bn2_scale, L0_0_bn2_shift)
    h = _block(h, L0_1_conv1, L0_1_bn1_scale, L0_1_bn1_shift,
               L0_1_conv2, L0_1_bn2_scale, L0_1_bn2_shift)
    h = _block(h, L0_2_conv1, L0_2_bn1_scale, L0_2_bn1_shift,
               L0_2_conv2, L0_2_bn2_scale, L0_2_bn2_shift)

    h = _block_ds(h, L1_0_conv1, L1_0_bn1_scale, L1_0_bn1_shift, L1_0_conv2,
                  L1_0_bn2_scale, L1_0_bn2_shift,
                  L1_0_ds_conv, L1_0_ds_bn_scale, L1_0_ds_bn_shift)
    for blk in [
        (L1_1_conv1, L1_1_bn1_scale, L1_1_bn1_shift, L1_1_conv2, L1_1_bn2_scale, L1_1_bn2_shift),
        (L1_2_conv1, L1_2_bn1_scale, L1_2_bn1_shift, L1_2_conv2, L1_2_bn2_scale, L1_2_bn2_shift),
        (L1_3_conv1, L1_3_bn1_scale, L1_3_bn1_shift, L1_3_conv2, L1_3_bn2_scale, L1_3_bn2_shift),
    ]:
        h = _block(h, *blk)
    s1 = h

    h = _block_ds(h, L2_0_conv1, L2_0_bn1_scale, L2_0_bn1_shift, L2_0_conv2,
                  L2_0_bn2_scale, L2_0_bn2_shift,
                  L2_0_ds_conv, L2_0_ds_bn_scale, L2_0_ds_bn_shift)
    for blk in [
        (L2_1_conv1, L2_1_bn1_scale, L2_1_bn1_shift, L2_1_conv2, L2_1_bn2_scale, L2_1_bn2_shift),
        (L2_2_conv1, L2_2_bn1_scale, L2_2_bn1_shift, L2_2_conv2, L2_2_bn2_scale, L2_2_bn2_shift),
        (L2_3_conv1, L2_3_bn1_scale, L2_3_bn1_shift, L2_3_conv2, L2_3_bn2_scale, L2_3_bn2_shift),
        (L2_4_conv1, L2_4_bn1_scale, L2_4_bn1_shift, L2_4_conv2, L2_4_bn2_scale, L2_4_bn2_shift),
        (L2_5_conv1, L2_5_bn1_scale, L2_5_bn1_shift, L2_5_conv2, L2_5_bn2_scale, L2_5_bn2_shift),
    ]:
        h = _block(h, *blk)
    s2 = h

    h = _block_ds(h, L3_0_conv1, L3_0_bn1_scale, L3_0_bn1_shift, L3_0_conv2,
                  L3_0_bn2_scale, L3_0_bn2_shift,
                  L3_0_ds_conv, L3_0_ds_bn_scale, L3_0_ds_bn_shift)
    for blk in [
        (L3_1_conv1, L3_1_bn1_scale, L3_1_bn1_shift, L3_1_conv2, L3_1_bn2_scale, L3_1_bn2_shift),
        (L3_2_conv1, L3_2_bn1_scale, L3_2_bn1_shift, L3_2_conv2, L3_2_bn2_scale, L3_2_bn2_shift),
    ]:
        h = _block(h, *blk)
    s3 = h

    t3 = _score(s3, scores1_w, scores1_b)
    t3 = _upsample(t3, upsample_2x, 2, 1)
    t2 = _score(s2, scores2_w, scores2_b, res=t3)
    t2 = _upsample(t2, upsample_4x, 2, 1)
    t1 = _score(s1, scores3_w, scores3_b, res=t2)
    out = _upsample(t1, upsample_8x, 8, 4)
    return jnp.transpose(out, (0, 3, 1, 2))


# speedup vs baseline: 12.8416x; 1.9218x over previous
import functools

import jax
import jax.numpy as jnp
from jax.experimental import pallas as pl
from jax.experimental.pallas import tpu as pltpu

BF = jnp.bfloat16
F32 = jnp.float32


def _w3(w):
    co, ci, _, _ = w.shape
    return jnp.transpose(w, (2, 3, 1, 0)).reshape(9, ci, co).astype(BF)


def _w1(w):
    co, ci, _, _ = w.shape
    return jnp.transpose(w.reshape(co, ci), (1, 0)).astype(BF)


def _row(v):
    return v.reshape(1, -1).astype(F32)


def _phases_t(x):
    N, H2, W2, C = x.shape
    H, W = H2 // 2, W2 // 2
    t = x.reshape(N, H, 2, W, 2, C).transpose(0, 2, 4, 1, 3, 5)
    return t.reshape(N, 4, H, W, C)


def _conv3x3(xp, w, OH, OW):
    Cin = xp.shape[2]
    acc = None
    for i in range(3):
        for j in range(3):
            t = jax.lax.slice(xp, (i, j, 0), (i + OH, j + OW, Cin))
            d = jnp.dot(t.reshape(OH * OW, Cin), w[3 * i + j],
                        preferred_element_type=F32)
            acc = d if acc is None else acc + d
    return acc


def _conv3x3_s2(pp, w, OH, OW):
    Cin = pp[0].shape[2]
    acc = None
    for i in range(3):
        pr, a = (i - 1) % 2, (i + 1) // 2
        for j in range(3):
            pc, b = (j - 1) % 2, (j + 1) // 2
            src = pp[2 * pr + pc]
            t = jax.lax.slice(src, (a, b, 0), (a + OH, b + OW, Cin))
            d = jnp.dot(t.reshape(OH * OW, Cin), w[3 * i + j],
                        preferred_element_type=F32)
            acc = d if acc is None else acc + d
    return acc


def _res_block_tail(x_f32_flat, y, w2, s2, h2, H, W, C):
    yp = jnp.pad(y.astype(BF).reshape(H, W, C), ((1, 1), (1, 1), (0, 0)))
    z = _conv3x3(yp, w2, H, W)
    z = z * s2 + h2 + x_f32_flat
    return jnp.maximum(z, 0.0).reshape(H, W, C)


def _rb_kernel(x_ref, w1_ref, s1_ref, h1_ref, w2_ref, s2_ref, h2_ref, o_ref,
               *, H, W, C):
    x = x_ref[0]
    xp = jnp.pad(x.astype(BF), ((1, 1), (1, 1), (0, 0)))
    y = _conv3x3(xp, w1_ref[...], H, W)
    y = jnp.maximum(y * s1_ref[...] + h1_ref[...], 0.0)
    o_ref[0] = _res_block_tail(x.reshape(H * W, C), y, w2_ref[...],
                               s2_ref[...], h2_ref[...], H, W, C)


def _stem_rb_kernel(s4_ref, w00_ref, w01_ref, w10_ref, w11_ref,
                    ss_ref, sh_ref,
                    w1_ref, s1_ref, h1_ref, w2_ref, s2_ref, h2_ref, o_ref,
                    *, H, W, C):
    s4 = s4_ref[0]
    ys = []
    for wref in (w00_ref, w01_ref, w10_ref, w11_ref):
        y = _conv3x3(s4, wref[...], H, W)
        y = jnp.maximum(y * ss_ref[...] + sh_ref[...], 0.0)
        ys.append(jnp.pad(y.reshape(H, W, C), ((1, 1), (1, 1), (0, 0))))
    m = None
    for i in range(3):
        pr, a = (i - 1) % 2, (i + 1) // 2
        for j in range(3):
            pc, b = (j - 1) % 2, (j + 1) // 2
            t = jax.lax.slice(ys[2 * pr + pc], (a, b, 0), (a + H, b + W, C))
            m = t if m is None else jnp.maximum(m, t)
    x = m
    xp = jnp.pad(x.astype(BF), ((1, 1), (1, 1), (0, 0)))
    y = _conv3x3(xp, w1_ref[...], H, W)
    y = jnp.maximum(y * s1_ref[...] + h1_ref[...], 0.0)
    o_ref[0] = _res_block_tail(x.reshape(H * W, C), y, w2_ref[...],
                               s2_ref[...], h2_ref[...], H, W, C)


def _rbd_kernel(p_ref, w1_ref, s1_ref, h1_ref, w2_ref, s2_ref, h2_ref,
                wd_ref, sd_ref, hd_ref, o_ref, *, H, W, C):
    pp = [jnp.pad(p_ref[0, k], ((1, 1), (1, 1), (0, 0))) for k in range(4)]
    Cin = pp[0].shape[2]
    y = _conv3x3_s2(pp, w1_ref[...], H, W)
    y = jnp.maximum(y * s1_ref[...] + h1_ref[...], 0.0)
    yp = jnp.pad(y.astype(BF).reshape(H, W, C), ((1, 1), (1, 1), (0, 0)))
    z = _conv3x3(yp, w2_ref[...], H, W)
    xs = jax.lax.slice(pp[0], (1, 1, 0), (1 + H, 1 + W, Cin))
    idn = jnp.dot(xs.reshape(H * W, Cin), wd_ref[...],
                  preferred_element_type=F32)
    idn = idn * sd_ref[...] + hd_ref[...]
    z = z * s2_ref[...] + h2_ref[...] + idn
    o_ref[0] = jnp.maximum(z, 0.0).reshape(H, W, C)


def _pcall(body, ins, specs, N, H, W, C):
    return pl.pallas_call(
        body,
        out_shape=jax.ShapeDtypeStruct((N, H, W, C), F32),
        grid_spec=pltpu.PrefetchScalarGridSpec(
            num_scalar_prefetch=0,
            grid=(N,),
            in_specs=specs,
            out_specs=pl.BlockSpec((1, H, W, C), lambda n: (n, 0, 0, 0)),
        ),
        compiler_params=pltpu.CompilerParams(
            dimension_semantics=("parallel",)),
    )(*ins)


def _wspecs(Cin, C):
    return [
        pl.BlockSpec((9, Cin, C), lambda n: (0, 0, 0)),
        pl.BlockSpec((1, C), lambda n: (0, 0)),
        pl.BlockSpec((1, C), lambda n: (0, 0)),
        pl.BlockSpec((9, C, C), lambda n: (0, 0, 0)),
        pl.BlockSpec((1, C), lambda n: (0, 0)),
        pl.BlockSpec((1, C), lambda n: (0, 0)),
    ]


def _block(x, w1, s1, h1, w2, s2, h2):
    N, H, W, C = x.shape
    ins = [x, _w3(w1), _row(s1), _row(h1), _w3(w2), _row(s2), _row(h2)]
    specs = ([pl.BlockSpec((1, H, W, C), lambda n: (n, 0, 0, 0))]
             + _wspecs(C, C))
    body = functools.partial(_rb_kernel, H=H, W=W, C=C)
    return _pcall(body, ins, specs, N, H, W, C)


def _block_ds(x, w1, s1, h1, w2, s2, h2, wd, sd, hd):
    N, Hin, Win, Cin = x.shape
    C = w1.shape[0]
    H, W = Hin // 2, Win // 2
    ph = _phases_t(x.astype(BF))
    ins = [ph, _w3(w1), _row(s1), _row(h1), _w3(w2), _row(s2), _row(h2),
           _w1(wd), _row(sd), _row(hd)]
    pspec = pl.BlockSpec((1, 4, H, W, Cin), lambda n: (n, 0, 0, 0, 0))
    specs = ([pspec] + _wspecs(Cin, C)
             + [pl.BlockSpec((Cin, C), lambda n: (0, 0)),
                pl.BlockSpec((1, C), lambda n: (0, 0)),
                pl.BlockSpec((1, C), lambda n: (0, 0))])
    body = functools.partial(_rbd_kernel, H=H, W=W, C=C)
    return _pcall(body, ins, specs, N, H, W, C)


def _stem_block(x, conv1, s, h, w1, s1, h1, w2, s2, h2):
    N = x.shape[0]
    s4 = x.astype(BF).reshape(N, 3, 64, 4, 64, 4)
    s4 = s4.transpose(0, 2, 4, 3, 5, 1).reshape(N, 64, 64, 48)
    s4 = jnp.pad(s4, ((0, 0), (1, 1), (1, 1), (0, 0)))

    wps = []
    for p in range(2):
        for q in range(2):
            w12 = jnp.pad(conv1, ((0, 0), (0, 0),
                                  (2 * p + 1, 4 - 2 * p),
                                  (2 * q + 1, 4 - 2 * q)))
            w12 = w12.reshape(64, 3, 3, 4, 3, 4)
            w12 = jnp.transpose(w12, (2, 4, 3, 5, 1, 0))
            wps.append(w12.reshape(9, 48, 64).astype(BF))

    ins = [s4] + wps + [_row(s), _row(h),
                        _w3(w1), _row(s1), _row(h1),
                        _w3(w2), _row(s2), _row(h2)]
    wp_spec = pl.BlockSpec((9, 48, 64), lambda n: (0, 0, 0))
    specs = ([pl.BlockSpec((1, 66, 66, 48), lambda n: (n, 0, 0, 0))]
             + [wp_spec] * 4
             + [pl.BlockSpec((1, 64), lambda n: (0, 0))] * 2
             + _wspecs(64, 64))
    body = functools.partial(_stem_rb_kernel, H=64, W=64, C=64)
    return _pcall(body, ins, specs, N, 64, 64, 64)


def _expand_mat(w, s, pad, out_len, in_len):
    filt = w[0, 0]
    g = filt[:, s - 1] / jnp.sqrt(filt[s - 1, s - 1])
    t = jnp.arange(out_len)
    q = (t + pad) // s
    r = (t + pad) % s
    j = jnp.arange(in_len + 2)[:, None]
    E = (jnp.where(j == q[None, :], g[r + s][None, :], 0.0)
         + jnp.where(j == q[None, :] + 1, g[r][None, :], 0.0))
    return E.astype(BF)


def _head_kernel(s3_ref, s2_ref, s1_ref,
                 w1_ref, b1_ref, w2_ref, b2_ref, w3_ref, b3_ref,
                 ew2_ref, eh2_ref, ew4_ref, eh4_ref, ew8_ref, eh8_ref,
                 o_ref, *, NC):
    def up(t, ew_ref, eh_ref, H, OH):
        tp = jnp.pad(t, ((0, 0), (1, 1), (1, 1))).astype(BF)
        y = jnp.dot(tp.reshape(NC * (H + 2), H + 2), ew_ref[...],
                    preferred_element_type=F32)
        y = y.astype(BF).reshape(NC, H + 2, OH)
        eh = eh_ref[...]
        return jnp.stack(
            [jnp.dot(eh, y[c], preferred_element_type=F32)
             for c in range(NC)], axis=0)

    t3 = jnp.dot(w1_ref[...], s3_ref[0], preferred_element_type=F32)
    t3 = (t3 + b1_ref[...]).reshape(NC, 8, 8)
    u2 = up(t3, ew2_ref, eh2_ref, 8, 16)

    t2 = jnp.dot(w2_ref[...], s2_ref[0], preferred_element_type=F32)
    t2 = (t2 + b2_ref[...]).reshape(NC, 16, 16) + u2
    u4 = up(t2, ew4_ref, eh4_ref, 16, 32)

    t1 = jnp.dot(w3_ref[...], s1_ref[0], preferred_element_type=F32)
    t1 = (t1 + b3_ref[...]).reshape(NC, 32, 32) + u4

    tp = jnp.pad(t1, ((0, 0), (1, 1), (1, 1))).astype(BF)
    y = jnp.dot(tp.reshape(NC * 34, 34), ew8_ref[...],
                preferred_element_type=F32)
    y = y.astype(BF).reshape(NC, 34, 256)
    eh = eh8_ref[...]
    for c in range(NC):
        o_ref[0, c] = jnp.dot(eh, y[c], preferred_element_type=F32)


def _head(s3, s2, s1, w1, b1, w2, b2, w3, b3, u2, u4, u8):
    N = s3.shape[0]
    NC = w1.shape[0]

    def cmaj(s):
        n, h, w, c = s.shape
        return jnp.transpose(s.astype(BF), (0, 3, 1, 2)).reshape(n, c, h * w)

    s3c, s2c, s1c = cmaj(s3), cmaj(s2), cmaj(s1)
    ew2 = _expand_mat(u2, 2, 1, 16, 8)
    ew4 = _expand_mat(u4, 2, 1, 32, 16)
    ew8 = _expand_mat(u8, 8, 4, 256, 32)
    ins = [s3c, s2c, s1c,
           w1.reshape(NC, 512).astype(BF), b1.reshape(NC, 1).astype(F32),
           w2.reshape(NC, 256).astype(BF), b2.reshape(NC, 1).astype(F32),
           w3.reshape(NC, 128).astype(BF), b3.reshape(NC, 1).astype(F32),
           ew2, ew2.T, ew4, ew4.T, ew8, ew8.T]
    specs = [
        pl.BlockSpec((1, 512, 64), lambda n: (n, 0, 0)),
        pl.BlockSpec((1, 256, 256), lambda n: (n, 0, 0)),
        pl.BlockSpec((1, 128, 1024), lambda n: (n, 0, 0)),
    ]
    for a in ins[3:]:
        specs.append(pl.BlockSpec(a.shape, lambda n: (0, 0)))
    return pl.pallas_call(
        functools.partial(_head_kernel, NC=NC),
        out_shape=jax.ShapeDtypeStruct((N, NC, 256, 256), F32),
        grid_spec=pltpu.PrefetchScalarGridSpec(
            num_scalar_prefetch=0,
            grid=(N,),
            in_specs=specs,
            out_specs=pl.BlockSpec((1, NC, 256, 256),
                                   lambda n: (n, 0, 0, 0)),
        ),
        compiler_params=pltpu.CompilerParams(
            dimension_semantics=("parallel",)),
    )(*ins)


def kernel(x, conv1, bn1_scale, bn1_shift, L0_0_conv1, L0_0_bn1_scale, L0_0_bn1_shift, L0_0_conv2, L0_0_bn2_scale, L0_0_bn2_shift, L0_1_conv1, L0_1_bn1_scale, L0_1_bn1_shift, L0_1_conv2, L0_1_bn2_scale, L0_1_bn2_shift, L0_2_conv1, L0_2_bn1_scale, L0_2_bn1_shift, L0_2_conv2, L0_2_bn2_scale, L0_2_bn2_shift, L1_0_conv1, L1_0_bn1_scale, L1_0_bn1_shift, L1_0_conv2, L1_0_bn2_scale, L1_0_bn2_shift, L1_0_ds_conv, L1_0_ds_bn_scale, L1_0_ds_bn_shift, L1_1_conv1, L1_1_bn1_scale, L1_1_bn1_shift, L1_1_conv2, L1_1_bn2_scale, L1_1_bn2_shift, L1_2_conv1, L1_2_bn1_scale, L1_2_bn1_shift, L1_2_conv2, L1_2_bn2_scale, L1_2_bn2_shift, L1_3_conv1, L1_3_bn1_scale, L1_3_bn1_shift, L1_3_conv2, L1_3_bn2_scale, L1_3_bn2_shift, L2_0_conv1, L2_0_bn1_scale, L2_0_bn1_shift, L2_0_conv2, L2_0_bn2_scale, L2_0_bn2_shift, L2_0_ds_conv, L2_0_ds_bn_scale, L2_0_ds_bn_shift, L2_1_conv1, L2_1_bn1_scale, L2_1_bn1_shift, L2_1_conv2, L2_1_bn2_scale, L2_1_bn2_shift, L2_2_conv1, L2_2_bn1_scale, L2_2_bn1_shift, L2_2_conv2, L2_2_bn2_scale, L2_2_bn2_shift, L2_3_conv1, L2_3_bn1_scale, L2_3_bn1_shift, L2_3_conv2, L2_3_bn2_scale, L2_3_bn2_shift, L2_4_conv1, L2_4_bn1_scale, L2_4_bn1_shift, L2_4_conv2, L2_4_bn2_scale, L2_4_bn2_shift, L2_5_conv1, L2_5_bn1_scale, L2_5_bn1_shift, L2_5_conv2, L2_5_bn2_scale, L2_5_bn2_shift, L3_0_conv1, L3_0_bn1_scale, L3_0_bn1_shift, L3_0_conv2, L3_0_bn2_scale, L3_0_bn2_shift, L3_0_ds_conv, L3_0_ds_bn_scale, L3_0_ds_bn_shift, L3_1_conv1, L3_1_bn1_scale, L3_1_bn1_shift, L3_1_conv2, L3_1_bn2_scale, L3_1_bn2_shift, L3_2_conv1, L3_2_bn1_scale, L3_2_bn1_shift, L3_2_conv2, L3_2_bn2_scale, L3_2_bn2_shift, scores1_w, scores1_b, scores2_w, scores2_b, scores3_w, scores3_b, upsample_8x, upsample_4x, upsample_2x):
    h = _stem_block(x, conv1, bn1_scale, bn1_shift,
                    L0_0_conv1, L0_0_bn1_scale, L0_0_bn1_shift,
                    L0_0_conv2, L0_0_bn2_scale, L0_0_bn2_shift)
    h = _block(h, L0_1_conv1, L0_1_bn1_scale, L0_1_bn1_shift,
               L0_1_conv2, L0_1_bn2_scale, L0_1_bn2_shift)
    h = _block(h, L0_2_conv1, L0_2_bn1_scale, L0_2_bn1_shift,
               L0_2_conv2, L0_2_bn2_scale, L0_2_bn2_shift)

    h = _block_ds(h, L1_0_conv1, L1_0_bn1_scale, L1_0_bn1_shift, L1_0_conv2,
                  L1_0_bn2_scale, L1_0_bn2_shift,
                  L1_0_ds_conv, L1_0_ds_bn_scale, L1_0_ds_bn_shift)
    for blk in [
        (L1_1_conv1, L1_1_bn1_scale, L1_1_bn1_shift, L1_1_conv2, L1_1_bn2_scale, L1_1_bn2_shift),
        (L1_2_conv1, L1_2_bn1_scale, L1_2_bn1_shift, L1_2_conv2, L1_2_bn2_scale, L1_2_bn2_shift),
        (L1_3_conv1, L1_3_bn1_scale, L1_3_bn1_shift, L1_3_conv2, L1_3_bn2_scale, L1_3_bn2_shift),
    ]:
        h = _block(h, *blk)
    s1 = h

    h = _block_ds(h, L2_0_conv1, L2_0_bn1_scale, L2_0_bn1_shift, L2_0_conv2,
                  L2_0_bn2_scale, L2_0_bn2_shift,
                  L2_0_ds_conv, L2_0_ds_bn_scale, L2_0_ds_bn_shift)
    for blk in [
        (L2_1_conv1, L2_1_bn1_scale, L2_1_bn1_shift, L2_1_conv2, L2_1_bn2_scale, L2_1_bn2_shift),
        (L2_2_conv1, L2_2_bn1_scale, L2_2_bn1_shift, L2_2_conv2, L2_2_bn2_scale, L2_2_bn2_shift),
        (L2_3_conv1, L2_3_bn1_scale, L2_3_bn1_shift, L2_3_conv2, L2_3_bn2_scale, L2_3_bn2_shift),
        (L2_4_conv1, L2_4_bn1_scale, L2_4_bn1_shift, L2_4_conv2, L2_4_bn2_scale, L2_4_bn2_shift),
        (L2_5_conv1, L2_5_bn1_scale, L2_5_bn1_shift, L2_5_conv2, L2_5_bn2_scale, L2_5_bn2_shift),
    ]:
        h = _block(h, *blk)
    s2 = h

    h = _block_ds(h, L3_0_conv1, L3_0_bn1_scale, L3_0_bn1_shift, L3_0_conv2,
                  L3_0_bn2_scale, L3_0_bn2_shift,
                  L3_0_ds_conv, L3_0_ds_bn_scale, L3_0_ds_bn_shift)
    for blk in [
        (L3_1_conv1, L3_1_bn1_scale, L3_1_bn1_shift, L3_1_conv2, L3_1_bn2_scale, L3_1_bn2_shift),
        (L3_2_conv1, L3_2_bn1_scale, L3_2_bn1_shift, L3_2_conv2, L3_2_bn2_scale, L3_2_bn2_shift),
    ]:
        h = _block(h, *blk)
    s3 = h

    return _head(s3, s2, s1, scores1_w, scores1_b, scores2_w, scores2_b,
                 scores3_w, scores3_b, upsample_2x, upsample_4x, upsample_8x)


# speedup vs baseline: 13.1105x; 1.0209x over previous
import functools

import jax
import jax.numpy as jnp
from jax.experimental import pallas as pl
from jax.experimental.pallas import tpu as pltpu

BF = jnp.bfloat16
F32 = jnp.float32


def _w3(w):
    co, ci, _, _ = w.shape
    return jnp.transpose(w, (2, 3, 1, 0)).reshape(9, ci, co).astype(BF)


def _w1(w):
    co, ci, _, _ = w.shape
    return jnp.transpose(w.reshape(co, ci), (1, 0)).astype(BF)


def _row(v):
    return v.reshape(1, -1).astype(F32)


def _phases_t(x):
    N, H2, W2, C = x.shape
    H, W = H2 // 2, W2 // 2
    t = x.reshape(N, H, 2, W, 2, C).transpose(0, 2, 4, 1, 3, 5)
    return t.reshape(N, 4, H, W, C)


def _conv3x3(xp, w, OH, OW):
    Cin = xp.shape[2]
    acc = None
    for i in range(3):
        for j in range(3):
            t = jax.lax.slice(xp, (i, j, 0), (i + OH, j + OW, Cin))
            d = jnp.dot(t.reshape(OH * OW, Cin), w[3 * i + j],
                        preferred_element_type=F32)
            acc = d if acc is None else acc + d
    return acc


def _conv3x3_s2(pp, w, OH, OW):
    Cin = pp[0].shape[2]
    acc = None
    for i in range(3):
        pr, a = (i - 1) % 2, (i + 1) // 2
        for j in range(3):
            pc, b = (j - 1) % 2, (j + 1) // 2
            src = pp[2 * pr + pc]
            t = jax.lax.slice(src, (a, b, 0), (a + OH, b + OW, Cin))
            d = jnp.dot(t.reshape(OH * OW, Cin), w[3 * i + j],
                        preferred_element_type=F32)
            acc = d if acc is None else acc + d
    return acc


def _res_block_tail(x_f32_flat, y, w2, s2, h2, H, W, C):
    yp = jnp.pad(y.astype(BF).reshape(H, W, C), ((1, 1), (1, 1), (0, 0)))
    z = _conv3x3(yp, w2, H, W)
    z = z * s2 + h2 + x_f32_flat
    return jnp.maximum(z, 0.0).reshape(H, W, C)


def _rb_kernel(x_ref, w1_ref, s1_ref, h1_ref, w2_ref, s2_ref, h2_ref, o_ref,
               *, H, W, C):
    x = x_ref[0]
    xp = jnp.pad(x.astype(BF), ((1, 1), (1, 1), (0, 0)))
    y = _conv3x3(xp, w1_ref[...], H, W)
    y = jnp.maximum(y * s1_ref[...] + h1_ref[...], 0.0)
    o_ref[0] = _res_block_tail(x.reshape(H * W, C), y, w2_ref[...],
                               s2_ref[...], h2_ref[...], H, W, C)


def _stem_rb_kernel(s4_ref, w00_ref, w01_ref, w10_ref, w11_ref,
                    ss_ref, sh_ref,
                    w1_ref, s1_ref, h1_ref, w2_ref, s2_ref, h2_ref, o_ref,
                    *, H, W, C):
    s4 = s4_ref[0]
    ys = []
    taps = ((0, 1), (0, 1, 2))
    for idx, wref in enumerate((w00_ref, w01_ref, w10_ref, w11_ref)):
        w = wref[...]
        acc = None
        for i in taps[idx // 2]:
            for j in taps[idx % 2]:
                t = jax.lax.slice(s4, (i, j, 0), (i + H, j + W, 48))
                d = jnp.dot(t.reshape(H * W, 48), w[3 * i + j],
                            preferred_element_type=F32)
                acc = d if acc is None else acc + d
        y = jnp.maximum(acc * ss_ref[...] + sh_ref[...], 0.0)
        ys.append(jnp.pad(y.reshape(H, W, C), ((1, 1), (1, 1), (0, 0))))
    m = None
    for i in range(3):
        pr, a = (i - 1) % 2, (i + 1) // 2
        for j in range(3):
            pc, b = (j - 1) % 2, (j + 1) // 2
            t = jax.lax.slice(ys[2 * pr + pc], (a, b, 0), (a + H, b + W, C))
            m = t if m is None else jnp.maximum(m, t)
    x = m
    xp = jnp.pad(x.astype(BF), ((1, 1), (1, 1), (0, 0)))
    y = _conv3x3(xp, w1_ref[...], H, W)
    y = jnp.maximum(y * s1_ref[...] + h1_ref[...], 0.0)
    o_ref[0] = _res_block_tail(x.reshape(H * W, C), y, w2_ref[...],
                               s2_ref[...], h2_ref[...], H, W, C)


def _rbd_kernel(p_ref, w1_ref, s1_ref, h1_ref, w2_ref, s2_ref, h2_ref,
                wd_ref, sd_ref, hd_ref, o_ref, *, H, W, C):
    pp = [jnp.pad(p_ref[0, k], ((1, 1), (1, 1), (0, 0))) for k in range(4)]
    Cin = pp[0].shape[2]
    y = _conv3x3_s2(pp, w1_ref[...], H, W)
    y = jnp.maximum(y * s1_ref[...] + h1_ref[...], 0.0)
    yp = jnp.pad(y.astype(BF).reshape(H, W, C), ((1, 1), (1, 1), (0, 0)))
    z = _conv3x3(yp, w2_ref[...], H, W)
    xs = jax.lax.slice(pp[0], (1, 1, 0), (1 + H, 1 + W, Cin))
    idn = jnp.dot(xs.reshape(H * W, Cin), wd_ref[...],
                  preferred_element_type=F32)
    idn = idn * sd_ref[...] + hd_ref[...]
    z = z * s2_ref[...] + h2_ref[...] + idn
    o_ref[0] = jnp.maximum(z, 0.0).reshape(H, W, C)


def _pcall(body, ins, specs, N, H, W, C):
    return pl.pallas_call(
        body,
        out_shape=jax.ShapeDtypeStruct((N, H, W, C), F32),
        grid_spec=pltpu.PrefetchScalarGridSpec(
            num_scalar_prefetch=0,
            grid=(N,),
            in_specs=specs,
            out_specs=pl.BlockSpec((1, H, W, C), lambda n: (n, 0, 0, 0)),
        ),
        compiler_params=pltpu.CompilerParams(
            dimension_semantics=("parallel",)),
    )(*ins)


def _wspecs(Cin, C):
    return [
        pl.BlockSpec((9, Cin, C), lambda n: (0, 0, 0)),
        pl.BlockSpec((1, C), lambda n: (0, 0)),
        pl.BlockSpec((1, C), lambda n: (0, 0)),
        pl.BlockSpec((9, C, C), lambda n: (0, 0, 0)),
        pl.BlockSpec((1, C), lambda n: (0, 0)),
        pl.BlockSpec((1, C), lambda n: (0, 0)),
    ]


def _block(x, w1, s1, h1, w2, s2, h2):
    N, H, W, C = x.shape
    ins = [x, _w3(w1), _row(s1), _row(h1), _w3(w2), _row(s2), _row(h2)]
    specs = ([pl.BlockSpec((1, H, W, C), lambda n: (n, 0, 0, 0))]
             + _wspecs(C, C))
    body = functools.partial(_rb_kernel, H=H, W=W, C=C)
    return _pcall(body, ins, specs, N, H, W, C)


def _block_ds(x, w1, s1, h1, w2, s2, h2, wd, sd, hd):
    N, Hin, Win, Cin = x.shape
    C = w1.shape[0]
    H, W = Hin // 2, Win // 2
    ph = _phases_t(x.astype(BF))
    ins = [ph, _w3(w1), _row(s1), _row(h1), _w3(w2), _row(s2), _row(h2),
           _w1(wd), _row(sd), _row(hd)]
    pspec = pl.BlockSpec((1, 4, H, W, Cin), lambda n: (n, 0, 0, 0, 0))
    specs = ([pspec] + _wspecs(Cin, C)
             + [pl.BlockSpec((Cin, C), lambda n: (0, 0)),
                pl.BlockSpec((1, C), lambda n: (0, 0)),
                pl.BlockSpec((1, C), lambda n: (0, 0))])
    body = functools.partial(_rbd_kernel, H=H, W=W, C=C)
    return _pcall(body, ins, specs, N, H, W, C)


def _stem_block(x, conv1, s, h, w1, s1, h1, w2, s2, h2):
    N = x.shape[0]
    s4 = x.astype(BF).reshape(N, 3, 64, 4, 64, 4)
    s4 = s4.transpose(0, 2, 4, 1, 3, 5).reshape(N, 64, 64, 48)
    s4 = jnp.pad(s4, ((0, 0), (1, 1), (1, 1), (0, 0)))

    wps = []
    for p in range(2):
        for q in range(2):
            w12 = jnp.pad(conv1, ((0, 0), (0, 0),
                                  (2 * p + 1, 4 - 2 * p),
                                  (2 * q + 1, 4 - 2 * q)))
            w12 = w12.reshape(64, 3, 3, 4, 3, 4)
            w12 = jnp.transpose(w12, (2, 4, 1, 3, 5, 0))
            wps.append(w12.reshape(9, 48, 64).astype(BF))

    ins = [s4] + wps + [_row(s), _row(h),
                        _w3(w1), _row(s1), _row(h1),
                        _w3(w2), _row(s2), _row(h2)]
    wp_spec = pl.BlockSpec((9, 48, 64), lambda n: (0, 0, 0))
    specs = ([pl.BlockSpec((1, 66, 66, 48), lambda n: (n, 0, 0, 0))]
             + [wp_spec] * 4
             + [pl.BlockSpec((1, 64), lambda n: (0, 0))] * 2
             + _wspecs(64, 64))
    body = functools.partial(_stem_rb_kernel, H=64, W=64, C=64)
    return _pcall(body, ins, specs, N, 64, 64, 64)


def _expand_mat(w, s, pad, out_len, in_len):
    filt = w[0, 0]
    g = filt[:, s - 1] / jnp.sqrt(filt[s - 1, s - 1])
    t = jnp.arange(out_len)
    q = (t + pad) // s
    r = (t + pad) % s
    j = jnp.arange(in_len + 2)[:, None]
    E = (jnp.where(j == q[None, :], g[r + s][None, :], 0.0)
         + jnp.where(j == q[None, :] + 1, g[r][None, :], 0.0))
    return E.astype(BF)


def _head_kernel(s3_ref, s2_ref, s1_ref,
                 w1_ref, b1_ref, w2_ref, b2_ref, w3_ref, b3_ref,
                 ew2_ref, eh2_ref, ew4_ref, eh4_ref, ew8_ref, eh8_ref,
                 o_ref, *, NC):
    def up(t, ew_ref, eh_ref, H, OH):
        tp = jnp.pad(t, ((0, 0), (1, 1), (1, 1))).astype(BF)
        y = jnp.dot(tp.reshape(NC * (H + 2), H + 2), ew_ref[...],
                    preferred_element_type=F32)
        y = y.astype(BF).reshape(NC, H + 2, OH)
        eh = eh_ref[...]
        return jnp.stack(
            [jnp.dot(eh, y[c], preferred_element_type=F32)
             for c in range(NC)], axis=0)

    t3 = jnp.dot(w1_ref[...], s3_ref[0], preferred_element_type=F32)
    t3 = (t3 + b1_ref[...]).reshape(NC, 8, 8)
    u2 = up(t3, ew2_ref, eh2_ref, 8, 16)

    t2 = jnp.dot(w2_ref[...], s2_ref[0], preferred_element_type=F32)
    t2 = (t2 + b2_ref[...]).reshape(NC, 16, 16) + u2
    u4 = up(t2, ew4_ref, eh4_ref, 16, 32)

    t1 = jnp.dot(w3_ref[...], s1_ref[0], preferred_element_type=F32)
    t1 = (t1 + b3_ref[...]).reshape(NC, 32, 32) + u4

    tp = jnp.pad(t1, ((0, 0), (1, 1), (1, 1))).astype(BF)
    y = jnp.dot(tp.reshape(NC * 34, 34), ew8_ref[...],
                preferred_element_type=F32)
    y = y.astype(BF).reshape(NC, 34, 256)
    eh = eh8_ref[...]
    for c in range(NC):
        o_ref[0, c] = jnp.dot(eh, y[c], preferred_element_type=F32)


def _head(s3, s2, s1, w1, b1, w2, b2, w3, b3, u2, u4, u8):
    N = s3.shape[0]
    NC = w1.shape[0]

    def cmaj(s):
        n, h, w, c = s.shape
        return jnp.transpose(s.astype(BF), (0, 3, 1, 2)).reshape(n, c, h * w)

    s3c, s2c, s1c = cmaj(s3), cmaj(s2), cmaj(s1)
    ew2 = _expand_mat(u2, 2, 1, 16, 8)
    ew4 = _expand_mat(u4, 2, 1, 32, 16)
    ew8 = _expand_mat(u8, 8, 4, 256, 32)
    ins = [s3c, s2c, s1c,
           w1.reshape(NC, 512).astype(BF), b1.reshape(NC, 1).astype(F32),
           w2.reshape(NC, 256).astype(BF), b2.reshape(NC, 1).astype(F32),
           w3.reshape(NC, 128).astype(BF), b3.reshape(NC, 1).astype(F32),
           ew2, ew2.T, ew4, ew4.T, ew8, ew8.T]
    specs = [
        pl.BlockSpec((1, 512, 64), lambda n: (n, 0, 0)),
        pl.BlockSpec((1, 256, 256), lambda n: (n, 0, 0)),
        pl.BlockSpec((1, 128, 1024), lambda n: (n, 0, 0)),
    ]
    for a in ins[3:]:
        specs.append(pl.BlockSpec(a.shape, lambda n: (0, 0)))
    return pl.pallas_call(
        functools.partial(_head_kernel, NC=NC),
        out_shape=jax.ShapeDtypeStruct((N, NC, 256, 256), F32),
        grid_spec=pltpu.PrefetchScalarGridSpec(
            num_scalar_prefetch=0,
            grid=(N,),
            in_specs=specs,
            out_specs=pl.BlockSpec((1, NC, 256, 256),
                                   lambda n: (n, 0, 0, 0)),
        ),
        compiler_params=pltpu.CompilerParams(
            dimension_semantics=("parallel",)),
    )(*ins)


def kernel(x, conv1, bn1_scale, bn1_shift, L0_0_conv1, L0_0_bn1_scale, L0_0_bn1_shift, L0_0_conv2, L0_0_bn2_scale, L0_0_bn2_shift, L0_1_conv1, L0_1_bn1_scale, L0_1_bn1_shift, L0_1_conv2, L0_1_bn2_scale, L0_1_bn2_shift, L0_2_conv1, L0_2_bn1_scale, L0_2_bn1_shift, L0_2_conv2, L0_2_bn2_scale, L0_2_bn2_shift, L1_0_conv1, L1_0_bn1_scale, L1_0_bn1_shift, L1_0_conv2, L1_0_bn2_scale, L1_0_bn2_shift, L1_0_ds_conv, L1_0_ds_bn_scale, L1_0_ds_bn_shift, L1_1_conv1, L1_1_bn1_scale, L1_1_bn1_shift, L1_1_conv2, L1_1_bn2_scale, L1_1_bn2_shift, L1_2_conv1, L1_2_bn1_scale, L1_2_bn1_shift, L1_2_conv2, L1_2_bn2_scale, L1_2_bn2_shift, L1_3_conv1, L1_3_bn1_scale, L1_3_bn1_shift, L1_3_conv2, L1_3_bn2_scale, L1_3_bn2_shift, L2_0_conv1, L2_0_bn1_scale, L2_0_bn1_shift, L2_0_conv2, L2_0_bn2_scale, L2_0_bn2_shift, L2_0_ds_conv, L2_0_ds_bn_scale, L2_0_ds_bn_shift, L2_1_conv1, L2_1_bn1_scale, L2_1_bn1_shift, L2_1_conv2, L2_1_bn2_scale, L2_1_bn2_shift, L2_2_conv1, L2_2_bn1_scale, L2_2_bn1_shift, L2_2_conv2, L2_2_bn2_scale, L2_2_bn2_shift, L2_3_conv1, L2_3_bn1_scale, L2_3_bn1_shift, L2_3_conv2, L2_3_bn2_scale, L2_3_bn2_shift, L2_4_conv1, L2_4_bn1_scale, L2_4_bn1_shift, L2_4_conv2, L2_4_bn2_scale, L2_4_bn2_shift, L2_5_conv1, L2_5_bn1_scale, L2_5_bn1_shift, L2_5_conv2, L2_5_bn2_scale, L2_5_bn2_shift, L3_0_conv1, L3_0_bn1_scale, L3_0_bn1_shift, L3_0_conv2, L3_0_bn2_scale, L3_0_bn2_shift, L3_0_ds_conv, L3_0_ds_bn_scale, L3_0_ds_bn_shift, L3_1_conv1, L3_1_bn1_scale, L3_1_bn1_shift, L3_1_conv2, L3_1_bn2_scale, L3_1_bn2_shift, L3_2_conv1, L3_2_bn1_scale, L3_2_bn1_shift, L3_2_conv2, L3_2_bn2_scale, L3_2_bn2_shift, scores1_w, scores1_b, scores2_w, scores2_b, scores3_w, scores3_b, upsample_8x, upsample_4x, upsample_2x):
    h = _stem_block(x, conv1, bn1_scale, bn1_shift,
                    L0_0_conv1, L0_0_bn1_scale, L0_0_bn1_shift,
                    L0_0_conv2, L0_0_bn2_scale, L0_0_bn2_shift)
    h = _block(h, L0_1_conv1, L0_1_bn1_scale, L0_1_bn1_shift,
               L0_1_conv2, L0_1_bn2_scale, L0_1_bn2_shift)
    h = _block(h, L0_2_conv1, L0_2_bn1_scale, L0_2_bn1_shift,
               L0_2_conv2, L0_2_bn2_scale, L0_2_bn2_shift)

    h = _block_ds(h, L1_0_conv1, L1_0_bn1_scale, L1_0_bn1_shift, L1_0_conv2,
                  L1_0_bn2_scale, L1_0_bn2_shift,
                  L1_0_ds_conv, L1_0_ds_bn_scale, L1_0_ds_bn_shift)
    for blk in [
        (L1_1_conv1, L1_1_bn1_scale, L1_1_bn1_shift, L1_1_conv2, L1_1_bn2_scale, L1_1_bn2_shift),
        (L1_2_conv1, L1_2_bn1_scale, L1_2_bn1_shift, L1_2_conv2, L1_2_bn2_scale, L1_2_bn2_shift),
        (L1_3_conv1, L1_3_bn1_scale, L1_3_bn1_shift, L1_3_conv2, L1_3_bn2_scale, L1_3_bn2_shift),
    ]:
        h = _block(h, *blk)
    s1 = h

    h = _block_ds(h, L2_0_conv1, L2_0_bn1_scale, L2_0_bn1_shift, L2_0_conv2,
                  L2_0_bn2_scale, L2_0_bn2_shift,
                  L2_0_ds_conv, L2_0_ds_bn_scale, L2_0_ds_bn_shift)
    for blk in [
        (L2_1_conv1, L2_1_bn1_scale, L2_1_bn1_shift, L2_1_conv2, L2_1_bn2_scale, L2_1_bn2_shift),
        (L2_2_conv1, L2_2_bn1_scale, L2_2_bn1_shift, L2_2_conv2, L2_2_bn2_scale, L2_2_bn2_shift),
        (L2_3_conv1, L2_3_bn1_scale, L2_3_bn1_shift, L2_3_conv2, L2_3_bn2_scale, L2_3_bn2_shift),
        (L2_4_conv1, L2_4_bn1_scale, L2_4_bn1_shift, L2_4_conv2, L2_4_bn2_scale, L2_4_bn2_shift),
        (L2_5_conv1, L2_5_bn1_scale, L2_5_bn1_shift, L2_5_conv2, L2_5_bn2_scale, L2_5_bn2_shift),
    ]:
        h = _block(h, *blk)
    s2 = h

    h = _block_ds(h, L3_0_conv1, L3_0_bn1_scale, L3_0_bn1_shift, L3_0_conv2,
                  L3_0_bn2_scale, L3_0_bn2_shift,
                  L3_0_ds_conv, L3_0_ds_bn_scale, L3_0_ds_bn_shift)
    for blk in [
        (L3_1_conv1, L3_1_bn1_scale, L3_1_bn1_shift, L3_1_conv2, L3_1_bn2_scale, L3_1_bn2_shift),
        (L3_2_conv1, L3_2_bn1_scale, L3_2_bn1_shift, L3_2_conv2, L3_2_bn2_scale, L3_2_bn2_shift),
    ]:
        h = _block(h, *blk)
    s3 = h

    return _head(s3, s2, s1, scores1_w, scores1_b, scores2_w, scores2_b,
                 scores3_w, scores3_b, upsample_2x, upsample_4x, upsample_8x)
